# baseline (speedup 1.0000x reference)
import functools

import jax
import jax.numpy as jnp
from jax import lax
from jax.experimental import pallas as pl
from jax.experimental.pallas import tpu as pltpu
from jax.experimental.pallas import tpu_sc as plsc

_NW = 32
_NBUF = 4
_PF = 2


def kernel(input_ids, word_embeddings, position_embeddings):
    B, S = input_ids.shape
    D = word_embeddings.shape[1]
    batches_per_w = B // _NW
    mesh = plsc.VectorSubcoreMesh(core_axis_name="c", subcore_axis_name="s")

    @functools.partial(
        pl.kernel,
        mesh=mesh,
        out_type=jax.ShapeDtypeStruct((B, S, D), jnp.float32),
        scratch_types=[
            pltpu.VMEM((batches_per_w * S,), jnp.int32),
            [pltpu.VMEM((S, D), jnp.float32) for _ in range(_NBUF)],
            pltpu.VMEM((S, D), jnp.float32),
            [pltpu.SemaphoreType.DMA for _ in range(_NBUF)],
            [pltpu.SemaphoreType.DMA for _ in range(_NBUF)],
        ],
        compiler_params=pltpu.CompilerParams(use_tc_tiling_on_sc=False),
    )
    def body(ids_hbm, word_hbm, pos_hbm, out_hbm, idx_v, rows, pos_v, gsems, ssems):
        wid = lax.axis_index("s") * 2 + lax.axis_index("c")
        base = wid * batches_per_w
        pltpu.sync_copy(
            ids_hbm.at[pl.ds(base * S, batches_per_w * S)], idx_v
        )
        pltpu.sync_copy(pos_hbm.at[pl.ds(0, S)], pos_v)

        def issue_gather(ci, b):
            pltpu.async_copy(
                word_hbm.at[idx_v.at[pl.ds(ci * S, S)]], rows[b], gsems[b]
            )

        def wait_gather(b):
            pltpu.make_async_copy(
                word_hbm.at[idx_v.at[pl.ds(0, S)]], rows[b], gsems[b]
            ).wait()

        def issue_scatter(ci, b):
            pltpu.async_copy(rows[b], out_hbm.at[base + ci], ssems[b])

        def wait_scatter(b):
            pltpu.make_async_copy(rows[b], out_hbm.at[base], ssems[b]).wait()

        issue_gather(0, 0)
        issue_gather(1, 1)

        @pl.loop(0, batches_per_w, step=_NBUF)
        def _(ci0):
            for b in range(_NBUF):
                ci = ci0 + b
                pb = (b + _PF) % _NBUF

                @pl.when(ci + _PF < batches_per_w)
                def _():
                    @pl.when(ci >= _PF)
                    def _():
                        wait_scatter(pb)

                    issue_gather(ci + _PF, pb)

                wait_gather(b)
                buf = rows[b]

                @plsc.parallel_loop(0, S, 1, unroll=8)
                def _(r):
                    for k in range(D // 16):
                        sl = pl.ds(k * 16, 16)
                        buf[r, sl] = buf[r, sl] + pos_v[r, sl]

                issue_scatter(ci, b)

        for b in range(_NBUF):
            wait_scatter(b)

    ids_flat = input_ids.reshape(-1)
    return body(ids_flat, word_embeddings, position_embeddings)
